# Initial kernel scaffold; baseline (speedup 1.0000x reference)
import functools

import jax
import jax.numpy as jnp
from jax import lax
from jax.experimental import pallas as pl
from jax.experimental.pallas import tpu as pltpu
from jax.experimental.pallas import tpu_sc as plsc

N = 10000
E = 320000
D = 128
H = 128
C = 64

NC = 2
NS = 16
NW = NC * NS
EPT = E // NW
K = 80
NCHUNK = EPT // K
RPT = N // NS

_MESH = plsc.VectorSubcoreMesh(core_axis_name="c", subcore_axis_name="s")


def _deg_partials(src3, dst3, zeros_n):
    @functools.partial(
        pl.kernel,
        out_type=jax.ShapeDtypeStruct((NC, 2, N), jnp.float32),
        mesh=_MESH,
        scratch_types=[
            pltpu.VMEM((NCHUNK, K), jnp.int32),
            pltpu.VMEM((NCHUNK, K), jnp.int32),
            pltpu.VMEM((K,), jnp.float32),
            pltpu.VMEM_SHARED((N,), jnp.float32),
            pltpu.VMEM_SHARED((N,), jnp.float32),
        ],
    )
    def body(src_hbm, dst_hbm, zeros_hbm, out_hbm, srcv, dstv, onesv, dsrc_sh, ddst_sh):
        c = lax.axis_index("c")
        s = lax.axis_index("s")
        wid = c * NS + s
        pltpu.sync_copy(src_hbm.at[wid], srcv)
        pltpu.sync_copy(dst_hbm.at[wid], dstv)
        for i in range(K // 16):
            onesv[pl.ds(i * 16, 16)] = jnp.ones((16,), jnp.float32)

        @pl.when(s == 0)
        def _zero():
            pltpu.sync_copy(zeros_hbm, dsrc_sh)
            pltpu.sync_copy(zeros_hbm, ddst_sh)

        plsc.subcore_barrier()

        def step(j, carry):
            pltpu.sync_copy(onesv, dsrc_sh.at[srcv.at[j]], add=True)
            pltpu.sync_copy(onesv, ddst_sh.at[dstv.at[j]], add=True)
            return carry

        lax.fori_loop(0, NCHUNK, step, 0)
        plsc.subcore_barrier()

        @pl.when(s == 0)
        def _out0():
            pltpu.sync_copy(dsrc_sh, out_hbm.at[c, 0])

        @pl.when(s == 1)
        def _out1():
            pltpu.sync_copy(ddst_sh, out_hbm.at[c, 1])

    return body(src3, dst3, zeros_n)


def _agg_partials(hw, src3, dst3, zeros_nf, F):
    @functools.partial(
        pl.kernel,
        out_type=jax.ShapeDtypeStruct((NC, N, F), jnp.float32),
        mesh=_MESH,
        scratch_types=[
            pltpu.VMEM((NCHUNK, K), jnp.int32),
            pltpu.VMEM((NCHUNK, K), jnp.int32),
            pltpu.VMEM((K, F), jnp.float32),
            pltpu.VMEM_SHARED((N, F), jnp.float32),
            pltpu.SemaphoreType.DMA,
        ],
    )
    def body(hw_hbm, src_hbm, dst_hbm, zeros_hbm, out_hbm, srcv, dstv, rows, acc_sh, sem):
        c = lax.axis_index("c")
        s = lax.axis_index("s")
        wid = c * NS + s
        pltpu.sync_copy(src_hbm.at[wid], srcv)
        pltpu.sync_copy(dst_hbm.at[wid], dstv)
        r0 = s * RPT
        pltpu.sync_copy(zeros_hbm.at[pl.ds(r0, RPT)], acc_sh.at[pl.ds(r0, RPT)])
        plsc.subcore_barrier()

        def step(j, carry):
            pltpu.async_copy(hw_hbm.at[srcv.at[j]], rows, sem).wait()
            pltpu.sync_copy(rows, acc_sh.at[dstv.at[j]], add=True)
            return carry

        lax.fori_loop(0, NCHUNK, step, 0)
        plsc.subcore_barrier()
        pltpu.sync_copy(acc_sh.at[pl.ds(r0, RPT)], out_hbm.at[c].at[pl.ds(r0, RPT)])

    return body(hw, src3, dst3, zeros_nf)


def _dense_first(degT, feat, W1):
    def body(degT_ref, feat_ref, w1_ref, ns_ref, nd_ref, hw_ref):
        deg_out = degT_ref[:, 0:1] + degT_ref[:, 2:3]
        deg_in = degT_ref[:, 1:2] + degT_ref[:, 3:4]
        ns = lax.rsqrt(jnp.maximum(deg_out, 1.0))
        nd = lax.rsqrt(jnp.maximum(deg_in, 1.0))
        ns_b = jnp.broadcast_to(ns, (N, H))
        nd_b = jnp.broadcast_to(nd, (N, H))
        ns_ref[...] = ns_b
        nd_ref[...] = nd_b
        hw_ref[...] = jnp.dot(feat_ref[...] * ns_b, w1_ref[...],
                              preferred_element_type=jnp.float32)

    return pl.pallas_call(
        body,
        out_shape=(
            jax.ShapeDtypeStruct((N, H), jnp.float32),
            jax.ShapeDtypeStruct((N, H), jnp.float32),
            jax.ShapeDtypeStruct((N, H), jnp.float32),
        ),
    )(degT, feat, W1)


def _dense_mid(aggp, ns_b, nd_b, W):
    Fo = W.shape[1]

    def body(aggp_ref, ns_ref, nd_ref, w_ref, out_ref):
        agg = aggp_ref[0] + aggp_ref[1]
        h = jax.nn.relu(agg * nd_ref[...]) * ns_ref[...]
        out_ref[...] = jnp.dot(h, w_ref[...], preferred_element_type=jnp.float32)

    return pl.pallas_call(
        body,
        out_shape=jax.ShapeDtypeStruct((N, Fo), jnp.float32),
    )(aggp, ns_b, nd_b, W)


def _dense_out(aggp, nd_b, b3):
    def body(aggp_ref, nd_ref, b_ref, out_ref):
        agg = aggp_ref[0] + aggp_ref[1]
        out_ref[...] = agg * nd_ref[:, :C] + b_ref[...][None, :]

    return pl.pallas_call(
        body,
        out_shape=jax.ShapeDtypeStruct((N, C), jnp.float32),
    )(aggp, nd_b, b3)


def kernel(feat, edge_index, W1, W2, W3, b3):
    src3 = edge_index[0].reshape(NW, NCHUNK, K)
    dst3 = edge_index[1].reshape(NW, NCHUNK, K)
    zn = jnp.zeros((N,), jnp.float32)
    z128 = jnp.zeros((N, H), jnp.float32)
    z64 = jnp.zeros((N, C), jnp.float32)

    degp = _deg_partials(src3, dst3, zn)
    degT = jnp.transpose(degp.reshape(2 * 2, N))
    ns_b, nd_b, hw1 = _dense_first(degT, feat, W1)
    aggp1 = _agg_partials(hw1, src3, dst3, z128, H)
    hw2 = _dense_mid(aggp1, ns_b, nd_b, W2)
    aggp2 = _agg_partials(hw2, src3, dst3, z128, H)
    hw3 = _dense_mid(aggp2, ns_b, nd_b, W3)
    aggp3 = _agg_partials(hw3, src3, dst3, z64, C)
    return _dense_out(aggp3, nd_b, b3)


# speedup vs baseline: 6.4911x; 6.4911x over previous
import functools

import jax
import jax.numpy as jnp
from jax import lax
from jax.experimental import pallas as pl
from jax.experimental.pallas import tpu as pltpu
from jax.experimental.pallas import tpu_sc as plsc

N = 10000
E = 320000
D = 128
H = 128
C = 64

NC = 2
NS = 16
NW = NC * NS
EPT = E // NW
K = 80
NCHUNK = EPT // K
R_A = 632
R_LAST = N - 15 * R_A

_MESH = plsc.VectorSubcoreMesh(core_axis_name="c", subcore_axis_name="s")


def _deg_partials(src3, dst3, ones_k8, zeros_n8):
    @functools.partial(
        pl.kernel,
        out_type=jax.ShapeDtypeStruct((NC, 2, N, 8), jnp.float32),
        mesh=_MESH,
        scratch_types=[
            pltpu.VMEM((NCHUNK, K), jnp.int32),
            pltpu.VMEM((NCHUNK, K), jnp.int32),
            pltpu.VMEM((K, 8), jnp.float32),
            pltpu.VMEM_SHARED((N, 8), jnp.float32),
            pltpu.VMEM_SHARED((N, 8), jnp.float32),
        ],
    )
    def body(src_hbm, dst_hbm, ones_hbm, zeros_hbm, out_hbm,
             srcv, dstv, onesv, dsrc_sh, ddst_sh):
        c = lax.axis_index("c")
        s = lax.axis_index("s")
        wid = c * NS + s
        pltpu.sync_copy(src_hbm.at[wid], srcv)
        pltpu.sync_copy(dst_hbm.at[wid], dstv)
        pltpu.sync_copy(ones_hbm, onesv)

        @pl.when(s == 0)
        def _zero_src():
            pltpu.sync_copy(zeros_hbm, dsrc_sh)

        @pl.when(s == 1)
        def _zero_dst():
            pltpu.sync_copy(zeros_hbm, ddst_sh)

        plsc.subcore_barrier()

        def step(j, carry):
            pltpu.sync_copy(onesv, dsrc_sh.at[srcv.at[j]], add=True)
            pltpu.sync_copy(onesv, ddst_sh.at[dstv.at[j]], add=True)
            return carry

        lax.fori_loop(0, NCHUNK, step, 0)
        plsc.subcore_barrier()

        @pl.when(s == 0)
        def _out0():
            pltpu.sync_copy(dsrc_sh, out_hbm.at[c, 0])

        @pl.when(s == 1)
        def _out1():
            pltpu.sync_copy(ddst_sh, out_hbm.at[c, 1])

    return body(src3, dst3, ones_k8, zeros_n8)


def _agg_partials(hw, src3, dst3, zeros_nf, F):
    @functools.partial(
        pl.kernel,
        out_type=jax.ShapeDtypeStruct((NC, N, F), jnp.float32),
        mesh=_MESH,
        scratch_types=[
            pltpu.VMEM((NCHUNK, K), jnp.int32),
            pltpu.VMEM((NCHUNK, K), jnp.int32),
            pltpu.VMEM((K, F), jnp.float32),
            pltpu.VMEM_SHARED((N, F), jnp.float32),
            pltpu.SemaphoreType.DMA,
        ],
    )
    def body(hw_hbm, src_hbm, dst_hbm, zeros_hbm, out_hbm, srcv, dstv, rows, acc_sh, sem):
        c = lax.axis_index("c")
        s = lax.axis_index("s")
        wid = c * NS + s
        pltpu.sync_copy(src_hbm.at[wid], srcv)
        pltpu.sync_copy(dst_hbm.at[wid], dstv)

        @pl.when(s < 15)
        def _zero_a():
            pltpu.sync_copy(zeros_hbm.at[pl.ds(s * R_A, R_A)],
                            acc_sh.at[pl.ds(s * R_A, R_A)])

        @pl.when(s == 15)
        def _zero_b():
            pltpu.sync_copy(zeros_hbm.at[pl.ds(15 * R_A, R_LAST)],
                            acc_sh.at[pl.ds(15 * R_A, R_LAST)])

        plsc.subcore_barrier()

        def step(j, carry):
            pltpu.async_copy(hw_hbm.at[srcv.at[j]], rows, sem).wait()
            pltpu.sync_copy(rows, acc_sh.at[dstv.at[j]], add=True)
            return carry

        lax.fori_loop(0, NCHUNK, step, 0)
        plsc.subcore_barrier()

        @pl.when(s < 15)
        def _out_a():
            pltpu.sync_copy(acc_sh.at[pl.ds(s * R_A, R_A)],
                            out_hbm.at[c].at[pl.ds(s * R_A, R_A)])

        @pl.when(s == 15)
        def _out_b():
            pltpu.sync_copy(acc_sh.at[pl.ds(15 * R_A, R_LAST)],
                            out_hbm.at[c].at[pl.ds(15 * R_A, R_LAST)])

    return body(hw, src3, dst3, zeros_nf)


def _dense_first(degT, feat, W1):
    def body(degT_ref, feat_ref, w1_ref, ns_ref, nd_ref, hw_ref):
        d = degT_ref[...]
        deg_out = d[0, :, 0:1] + d[2, :, 0:1]
        deg_in = d[1, :, 0:1] + d[3, :, 0:1]
        ns = lax.rsqrt(jnp.maximum(deg_out, 1.0))
        nd = lax.rsqrt(jnp.maximum(deg_in, 1.0))
        ns_b = jnp.broadcast_to(ns, (N, H))
        nd_b = jnp.broadcast_to(nd, (N, H))
        ns_ref[...] = ns_b
        nd_ref[...] = nd_b
        hw_ref[...] = jnp.dot(feat_ref[...] * ns_b, w1_ref[...],
                              preferred_element_type=jnp.float32)

    return pl.pallas_call(
        body,
        out_shape=(
            jax.ShapeDtypeStruct((N, H), jnp.float32),
            jax.ShapeDtypeStruct((N, H), jnp.float32),
            jax.ShapeDtypeStruct((N, H), jnp.float32),
        ),
    )(degT, feat, W1)


def _dense_mid(aggp, ns_b, nd_b, W):
    Fo = W.shape[1]

    def body(aggp_ref, ns_ref, nd_ref, w_ref, out_ref):
        agg = aggp_ref[0] + aggp_ref[1]
        h = jax.nn.relu(agg * nd_ref[...]) * ns_ref[...]
        out_ref[...] = jnp.dot(h, w_ref[...], preferred_element_type=jnp.float32)

    return pl.pallas_call(
        body,
        out_shape=jax.ShapeDtypeStruct((N, Fo), jnp.float32),
    )(aggp, ns_b, nd_b, W)


def _dense_act(aggp, ns_b, nd_b):
    def body(aggp_ref, ns_ref, nd_ref, out_ref):
        agg = aggp_ref[0] + aggp_ref[1]
        out_ref[...] = jax.nn.relu(agg * nd_ref[...]) * ns_ref[...]

    return pl.pallas_call(
        body,
        out_shape=jax.ShapeDtypeStruct((N, H), jnp.float32),
    )(aggp, ns_b, nd_b)


def _dense_out(aggp, nd_b, W3, b3):
    def body(aggp_ref, nd_ref, w_ref, b_ref, out_ref):
        agg = aggp_ref[0] + aggp_ref[1]
        mm = jnp.dot(agg, w_ref[...], preferred_element_type=jnp.float32)
        out_ref[...] = mm * nd_ref[:, :C] + b_ref[...][None, :]

    return pl.pallas_call(
        body,
        out_shape=jax.ShapeDtypeStruct((N, C), jnp.float32),
    )(aggp, nd_b, W3, b3)


def kernel(feat, edge_index, W1, W2, W3, b3):
    src3 = edge_index[0].reshape(NW, NCHUNK, K)
    dst3 = edge_index[1].reshape(NW, NCHUNK, K)
    zn8 = jnp.zeros((N, 8), jnp.float32)
    ones_k8 = jnp.ones((K, 8), jnp.float32)
    z128 = jnp.zeros((N, H), jnp.float32)

    degp = _deg_partials(src3, dst3, ones_k8, zn8)
    degT = degp.reshape(2 * 2, N, 8)
    ns_b, nd_b, hw1 = _dense_first(degT, feat, W1)
    aggp1 = _agg_partials(hw1, src3, dst3, z128, H)
    hw2 = _dense_mid(aggp1, ns_b, nd_b, W2)
    aggp2 = _agg_partials(hw2, src3, dst3, z128, H)
    u3 = _dense_act(aggp2, ns_b, nd_b)
    aggp3 = _agg_partials(u3, src3, dst3, z128, H)
    return _dense_out(aggp3, nd_b, W3, b3)
